# baseline (speedup 1.0000x reference)
import functools
import numpy as np
import jax
import jax.numpy as jnp
from jax import lax
from jax.experimental import pallas as pl
from jax.experimental.pallas import tpu as pltpu
from jax.experimental.pallas import tpu_sc as plsc

_VOCAB = 100000
_DIM = 64
_BATCH = 4096
_SEQ = 200

_NC = 2
_NS = 16
_L = 16
_NW = _NC * _NS
_BI = 128
_ST = _SEQ // 8
_NBUF = 5
_LEAD = 4
_NGROUP = _SEQ // _NBUF


def _positional_encoding_np(seq, d_model):
    pos = np.arange(seq, dtype=np.float32)[:, None]
    i = np.arange(0, d_model, 2, dtype=np.float32)
    div = np.power(10000.0, i / d_model)
    pe = np.zeros((seq, d_model), dtype=np.float32)
    pe[:, 0::2] = np.sin(pos / div)
    pe[:, 1::2] = np.cos(pos / div)
    return pe


_PE = _positional_encoding_np(_SEQ, _DIM)


@functools.partial(
    pl.kernel,
    out_type=jax.ShapeDtypeStruct((_SEQ, 8, _NW, 8, _BI), jnp.float32),
    mesh=plsc.VectorSubcoreMesh(core_axis_name="c", subcore_axis_name="s"),
    scratch_types=[
        pltpu.VMEM((_ST, 8, _BI), jnp.int32),
        pltpu.VMEM((_SEQ, _DIM), jnp.float32),
        pltpu.VMEM((_NBUF, _BI, _DIM), jnp.float32),
        pltpu.VMEM((_NBUF, 8, 8, _BI + 1), jnp.float32),
    ]
    + [pltpu.SemaphoreType.DMA] * (2 * _NBUF),
    compiler_params=pltpu.CompilerParams(
        use_tc_tiling_on_sc=False, needs_layout_passes=False
    ),
)
def _embed_kernel(x_hbm, pe_hbm, table_hbm, out_hbm, idx_v, pe_v, g_v, t_v, *sems):
    gsems = sems[:_NBUF]
    ssems = sems[_NBUF:]
    wid = lax.axis_index("s") * _NC + lax.axis_index("c")
    pltpu.sync_copy(x_hbm.at[:, wid], idx_v)
    pltpu.sync_copy(pe_hbm, pe_v)

    lanes = jnp.arange(_L, dtype=jnp.int32)
    di_idx = lanes % 8
    dt_idx = [dh * 2 + lanes // 8 for dh in range(_DIM // _L)]

    def gather_start(s, b):
        pltpu.async_copy(
            table_hbm.at[idx_v.at[s // 8, s % 8]], g_v.at[b], gsems[b]
        )

    def gather_wait(s, b):
        pltpu.make_async_copy(
            table_hbm.at[idx_v.at[s // 8, s % 8]], g_v.at[b], gsems[b]
        ).wait()

    def store_start(s, b):
        pltpu.async_copy(
            t_v.at[b, :, :, pl.ds(0, _BI)], out_hbm.at[s, :, wid], ssems[b]
        )

    def store_wait(b):
        pltpu.make_async_copy(
            t_v.at[b, :, :, pl.ds(0, _BI)], out_hbm.at[0, :, wid], ssems[b]
        ).wait()

    def transpose_pe(s, b):
        pe_row = [pe_v[s, pl.ds(dh * _L, _L)] for dh in range(_DIM // _L)]

        @plsc.parallel_loop(0, _BI, 1, unroll=4)
        def _(i):
            bi = jnp.full((_L,), i, dtype=jnp.int32)
            for dh in range(_DIM // _L):
                plsc.store_scatter(
                    t_v.at[b],
                    [dt_idx[dh], di_idx, bi],
                    g_v[b, i, pl.ds(dh * _L, _L)] + pe_row[dh],
                )

    def do_unit(s, b, first, last):
        gather_wait(s, b)
        if not first:
            store_wait(b)
        transpose_pe(s, b)
        store_start(s, b)
        if not last:
            gather_start(s + _LEAD, (b + _LEAD) % _NBUF)

    for b in range(_LEAD):
        gather_start(jnp.int32(b), b)

    for b in range(_NBUF):
        do_unit(jnp.int32(b), b, first=True, last=False)

    def group_body(g, carry):
        s0 = g * _NBUF
        for b in range(_NBUF):
            do_unit(s0 + b, b, first=False, last=False)
        return carry

    lax.fori_loop(1, _NGROUP - 1, group_body, 0)

    s0 = (_NGROUP - 1) * _NBUF
    for b in range(_NBUF):
        do_unit(jnp.int32(s0 + b), b, first=False, last=(b >= _NBUF - _LEAD))

    for b in range(_NBUF):
        store_wait(b)


def kernel(x, table):
    x4 = x.T.reshape(_ST, 8, _NW, _BI).transpose(0, 2, 1, 3)
    t2 = lax.optimization_barrier(table.reshape(_VOCAB // 2, 2 * _DIM))
    out = _embed_kernel(x4, _PE, t2.reshape(_VOCAB, _DIM))
    return out.transpose(2, 4, 0, 1, 3).reshape(_BATCH, _SEQ, _DIM)


# speedup vs baseline: 1.0009x; 1.0009x over previous
import functools
import numpy as np
import jax
import jax.numpy as jnp
from jax import lax
from jax.experimental import pallas as pl
from jax.experimental.pallas import tpu as pltpu
from jax.experimental.pallas import tpu_sc as plsc

_VOCAB = 100000
_DIM = 64
_BATCH = 4096
_SEQ = 200

_NC = 2
_NS = 16
_L = 16
_NW = _NC * _NS
_BI = 128
_ST = _SEQ // 8
_NBUF = 5
_LEAD = 4
_NGROUP = _SEQ // _NBUF


def _positional_encoding_np(seq, d_model):
    pos = np.arange(seq, dtype=np.float32)[:, None]
    i = np.arange(0, d_model, 2, dtype=np.float32)
    div = np.power(10000.0, i / d_model)
    pe = np.zeros((seq, d_model), dtype=np.float32)
    pe[:, 0::2] = np.sin(pos / div)
    pe[:, 1::2] = np.cos(pos / div)
    return pe


_PE = _positional_encoding_np(_SEQ, _DIM)


@functools.partial(
    pl.kernel,
    out_type=jax.ShapeDtypeStruct((_SEQ, 8, _NW, 8, _BI), jnp.float32),
    mesh=plsc.VectorSubcoreMesh(core_axis_name="c", subcore_axis_name="s"),
    scratch_types=[
        pltpu.VMEM((_ST, 8, _BI), jnp.int32),
        pltpu.VMEM((_SEQ, _DIM), jnp.float32),
        pltpu.VMEM((_NBUF, _BI, _DIM), jnp.float32),
        pltpu.VMEM((_NBUF, 8, 8, _BI + 1), jnp.float32),
    ]
    + [pltpu.SemaphoreType.DMA] * (2 * _NBUF),
    compiler_params=pltpu.CompilerParams(
        use_tc_tiling_on_sc=False, needs_layout_passes=False
    ),
)
def _embed_kernel(x_hbm, pe_hbm, table_hbm, out_hbm, idx_v, pe_v, g_v, t_v, *sems):
    gsems = sems[:_NBUF]
    ssems = sems[_NBUF:]
    wid = lax.axis_index("s") * _NC + lax.axis_index("c")
    pltpu.sync_copy(x_hbm.at[:, wid], idx_v)
    pltpu.sync_copy(pe_hbm, pe_v)

    lanes = jnp.arange(_L, dtype=jnp.int32)
    di_idx = lanes % 8
    dt_idx = [dh * 2 + lanes // 8 for dh in range(_DIM // _L)]

    def gather_start(s, b):
        pltpu.async_copy(
            table_hbm.at[idx_v.at[s // 8, s % 8]], g_v.at[b], gsems[b]
        )

    def gather_wait(s, b):
        pltpu.make_async_copy(
            table_hbm.at[idx_v.at[s // 8, s % 8]], g_v.at[b], gsems[b]
        ).wait()

    def store_start(s, b):
        pltpu.async_copy(
            t_v.at[b, :, :, pl.ds(0, _BI)], out_hbm.at[s, :, wid], ssems[b]
        )

    def store_wait(b):
        pltpu.make_async_copy(
            t_v.at[b, :, :, pl.ds(0, _BI)], out_hbm.at[0, :, wid], ssems[b]
        ).wait()

    def transpose_pe(s, b):
        pe_row = [pe_v[s, pl.ds(dh * _L, _L)] for dh in range(_DIM // _L)]

        @plsc.parallel_loop(0, _BI, 1, unroll=4)
        def _(i):
            bi = jnp.full((_L,), i, dtype=jnp.int32)
            for dh in range(_DIM // _L):
                plsc.store_scatter(
                    t_v.at[b],
                    [dt_idx[dh], di_idx, bi],
                    g_v[b, i, pl.ds(dh * _L, _L)] + pe_row[dh],
                )

    def do_unit(s, b, first, last):
        gather_wait(s, b)
        if not first:
            store_wait(b)
        transpose_pe(s, b)
        store_start(s, b)
        if not last:
            gather_start(s + _LEAD, (b + _LEAD) % _NBUF)

    for b in range(_LEAD):
        gather_start(jnp.int32(b), b)

    for b in range(_NBUF):
        do_unit(jnp.int32(b), b, first=True, last=False)

    def group_body(g, carry):
        s0 = g * _NBUF
        for b in range(_NBUF):
            do_unit(s0 + b, b, first=False, last=False)
        return carry

    lax.fori_loop(1, _NGROUP - 1, group_body, 0)

    s0 = (_NGROUP - 1) * _NBUF
    for b in range(_NBUF):
        do_unit(jnp.int32(s0 + b), b, first=False, last=(b >= _NBUF - _LEAD))

    for b in range(_NBUF):
        store_wait(b)


def kernel(x, table):
    x4 = x.T.reshape(_ST, 8, _NW, _BI).transpose(0, 2, 1, 3)
    out = _embed_kernel(x4, _PE, table)
    return out.transpose(2, 4, 0, 1, 3).reshape(_BATCH, _SEQ, _DIM)


# speedup vs baseline: 1.0036x; 1.0027x over previous
import functools
import numpy as np
import jax
import jax.numpy as jnp
from jax import lax
from jax.experimental import pallas as pl
from jax.experimental.pallas import tpu as pltpu
from jax.experimental.pallas import tpu_sc as plsc

_VOCAB = 100000
_DIM = 64
_BATCH = 4096
_SEQ = 200

_NC = 2
_NS = 16
_L = 16
_NW = _NC * _NS
_BI = 128
_ST = _SEQ // 8
_NBUF = 5
_LEAD = 5
_NGROUP = _SEQ // _NBUF


def _positional_encoding_np(seq, d_model):
    pos = np.arange(seq, dtype=np.float32)[:, None]
    i = np.arange(0, d_model, 2, dtype=np.float32)
    div = np.power(10000.0, i / d_model)
    pe = np.zeros((seq, d_model), dtype=np.float32)
    pe[:, 0::2] = np.sin(pos / div)
    pe[:, 1::2] = np.cos(pos / div)
    return pe


_PE = _positional_encoding_np(_SEQ, _DIM)


@functools.partial(
    pl.kernel,
    out_type=jax.ShapeDtypeStruct((_SEQ, 8, _NW, 8, _BI), jnp.float32),
    mesh=plsc.VectorSubcoreMesh(core_axis_name="c", subcore_axis_name="s"),
    scratch_types=[
        pltpu.VMEM((_ST, 8, _BI), jnp.int32),
        pltpu.VMEM((_SEQ, _DIM), jnp.float32),
        pltpu.VMEM((_NBUF, _BI, _DIM), jnp.float32),
        pltpu.VMEM((_NBUF, 8, 8, _BI + 1), jnp.float32),
    ]
    + [pltpu.SemaphoreType.DMA] * (2 * _NBUF),
    compiler_params=pltpu.CompilerParams(
        use_tc_tiling_on_sc=False, needs_layout_passes=False
    ),
)
def _embed_kernel(x_hbm, pe_hbm, table_hbm, out_hbm, idx_v, pe_v, g_v, t_v, *sems):
    gsems = sems[:_NBUF]
    ssems = sems[_NBUF:]
    wid = lax.axis_index("s") * _NC + lax.axis_index("c")
    pltpu.sync_copy(x_hbm.at[:, wid], idx_v)
    pltpu.sync_copy(pe_hbm, pe_v)

    lanes = jnp.arange(_L, dtype=jnp.int32)
    di_idx = lanes % 8
    dt_idx = [dh * 2 + lanes // 8 for dh in range(_DIM // _L)]

    def gather_start(s, b):
        pltpu.async_copy(
            table_hbm.at[idx_v.at[s // 8, s % 8]], g_v.at[b], gsems[b]
        )

    def gather_wait(s, b):
        pltpu.make_async_copy(
            table_hbm.at[idx_v.at[s // 8, s % 8]], g_v.at[b], gsems[b]
        ).wait()

    def store_start(s, b):
        pltpu.async_copy(
            t_v.at[b, :, :, pl.ds(0, _BI)], out_hbm.at[s, :, wid], ssems[b]
        )

    def store_wait(b):
        pltpu.make_async_copy(
            t_v.at[b, :, :, pl.ds(0, _BI)], out_hbm.at[0, :, wid], ssems[b]
        ).wait()

    def transpose_pe(s, b):
        pe_row = [pe_v[s, pl.ds(dh * _L, _L)] for dh in range(_DIM // _L)]

        @plsc.parallel_loop(0, _BI, 1, unroll=4)
        def _(i):
            bi = jnp.full((_L,), i, dtype=jnp.int32)
            for dh in range(_DIM // _L):
                plsc.store_scatter(
                    t_v.at[b],
                    [dt_idx[dh], di_idx, bi],
                    g_v[b, i, pl.ds(dh * _L, _L)] + pe_row[dh],
                )

    def do_unit(s, b, first, last):
        gather_wait(s, b)
        if not first:
            store_wait(b)
        transpose_pe(s, b)
        store_start(s, b)
        if not last:
            gather_start(s + _LEAD, (b + _LEAD) % _NBUF)

    for b in range(_LEAD):
        gather_start(jnp.int32(b), b)

    for b in range(_NBUF):
        do_unit(jnp.int32(b), b, first=True, last=False)

    def group_body(g, carry):
        s0 = g * _NBUF
        for b in range(_NBUF):
            do_unit(s0 + b, b, first=False, last=False)
        return carry

    lax.fori_loop(1, _NGROUP - 1, group_body, 0)

    s0 = (_NGROUP - 1) * _NBUF
    for b in range(_NBUF):
        do_unit(jnp.int32(s0 + b), b, first=False, last=(b >= _NBUF - _LEAD))

    for b in range(_NBUF):
        store_wait(b)


def kernel(x, table):
    x4 = x.T.reshape(_ST, 8, _NW, _BI).transpose(0, 2, 1, 3)
    out = _embed_kernel(x4, _PE, table)
    return out.transpose(2, 4, 0, 1, 3).reshape(_BATCH, _SEQ, _DIM)


# speedup vs baseline: 1.0051x; 1.0015x over previous
import functools
import numpy as np
import jax
import jax.numpy as jnp
from jax import lax
from jax.experimental import pallas as pl
from jax.experimental.pallas import tpu as pltpu
from jax.experimental.pallas import tpu_sc as plsc

_VOCAB = 100000
_DIM = 64
_BATCH = 4096
_SEQ = 200

_NC = 2
_NS = 16
_L = 16
_NW = _NC * _NS
_BI = 128
_ST = _SEQ // 8
_NBUF = 5
_LEAD = 5
_NGROUP = _SEQ // _NBUF


def _positional_encoding_np(seq, d_model):
    pos = np.arange(seq, dtype=np.float32)[:, None]
    i = np.arange(0, d_model, 2, dtype=np.float32)
    div = np.power(10000.0, i / d_model)
    pe = np.zeros((seq, d_model), dtype=np.float32)
    pe[:, 0::2] = np.sin(pos / div)
    pe[:, 1::2] = np.cos(pos / div)
    return pe


_PE = _positional_encoding_np(_SEQ, _DIM)


@functools.partial(
    pl.kernel,
    out_type=jax.ShapeDtypeStruct((_SEQ, 8, _NW, 8, _BI), jnp.float32),
    mesh=plsc.VectorSubcoreMesh(core_axis_name="c", subcore_axis_name="s"),
    scratch_types=[
        pltpu.VMEM((_ST, 8, _BI), jnp.int32),
        pltpu.VMEM((_SEQ, _DIM), jnp.float32),
        pltpu.VMEM((_NBUF, _BI, _DIM), jnp.float32),
        pltpu.VMEM((_NBUF, 8, 8, _BI + 1), jnp.float32),
    ]
    + [pltpu.SemaphoreType.DMA] * (2 * _NBUF),
    compiler_params=pltpu.CompilerParams(
        use_tc_tiling_on_sc=False, needs_layout_passes=False
    ),
)
def _embed_kernel(x_hbm, pe_hbm, table_hbm, out_hbm, idx_v, pe_v, g_v, t_v, *sems):
    gsems = sems[:_NBUF]
    ssems = sems[_NBUF:]
    wid = lax.axis_index("s") * _NC + lax.axis_index("c")
    pltpu.sync_copy(x_hbm.at[:, wid], idx_v)
    pltpu.sync_copy(pe_hbm, pe_v)

    lanes = jnp.arange(_L, dtype=jnp.int32)
    di_idx = lanes % 8
    dt_idx = [dh * 2 + lanes // 8 for dh in range(_DIM // _L)]

    def gather_start(s, b):
        pltpu.async_copy(
            table_hbm.at[idx_v.at[s // 8, s % 8]], g_v.at[b], gsems[b]
        )

    def gather_wait(s, b):
        pltpu.make_async_copy(
            table_hbm.at[idx_v.at[s // 8, s % 8]], g_v.at[b], gsems[b]
        ).wait()

    def store_start(s, b):
        pltpu.async_copy(
            t_v.at[b, :, :, pl.ds(0, _BI)], out_hbm.at[s, :, wid], ssems[b]
        )

    def store_wait(b):
        pltpu.make_async_copy(
            t_v.at[b, :, :, pl.ds(0, _BI)], out_hbm.at[0, :, wid], ssems[b]
        ).wait()

    def transpose_pe(s, b):
        pe_row = [pe_v[s, pl.ds(dh * _L, _L)] for dh in range(_DIM // _L)]

        @plsc.parallel_loop(0, _BI, 1, unroll=2)
        def _(i):
            bi = jnp.full((_L,), i, dtype=jnp.int32)
            for dh in range(_DIM // _L):
                plsc.store_scatter(
                    t_v.at[b],
                    [dt_idx[dh], di_idx, bi],
                    g_v[b, i, pl.ds(dh * _L, _L)] + pe_row[dh],
                )

    def do_unit(s, b, first, last):
        gather_wait(s, b)
        if not first:
            store_wait(b)
        transpose_pe(s, b)
        store_start(s, b)
        if not last:
            gather_start(s + _LEAD, (b + _LEAD) % _NBUF)

    for b in range(_LEAD):
        gather_start(jnp.int32(b), b)

    for b in range(_NBUF):
        do_unit(jnp.int32(b), b, first=True, last=False)

    def group_body(g, carry):
        s0 = g * _NBUF
        for b in range(_NBUF):
            do_unit(s0 + b, b, first=False, last=False)
        return carry

    lax.fori_loop(1, _NGROUP - 1, group_body, 0)

    s0 = (_NGROUP - 1) * _NBUF
    for b in range(_NBUF):
        do_unit(jnp.int32(s0 + b), b, first=False, last=(b >= _NBUF - _LEAD))

    for b in range(_NBUF):
        store_wait(b)


def kernel(x, table):
    x4 = x.T.reshape(_ST, 8, _NW, _BI).transpose(0, 2, 1, 3)
    out = _embed_kernel(x4, _PE, table)
    return out.transpose(2, 4, 0, 1, 3).reshape(_BATCH, _SEQ, _DIM)
